# baseline (speedup 1.0000x reference)
import functools

import jax
import jax.numpy as jnp
from jax import lax
from jax.experimental import pallas as pl
from jax.experimental.pallas import tpu as pltpu
from jax.experimental.pallas import tpu_sc as plsc

B, S, D = 4, 4096, 1024
NC, NS, L = 2, 16, 16
NW = NC * NS
P_PER_W = S // NW
C = 8
N_CHUNKS = P_PER_W // C
D_VREGS = D // L


_mesh = plsc.VectorSubcoreMesh(core_axis_name="c", subcore_axis_name="s")


@functools.partial(
    pl.kernel,
    mesh=_mesh,
    out_type=jax.ShapeDtypeStruct((B, S, D), jnp.float32),
    scratch_types=[
        pltpu.VMEM((C, D), jnp.float32),
        pltpu.VMEM((B, C, D), jnp.float32),
        pltpu.SemaphoreType.DMA,
    ],
)
def _pos_encode_sc(emb_hbm, pos_hbm, out_hbm, pos_v, emb_v, sem):
    wid = lax.axis_index("s") * NC + lax.axis_index("c")
    base = wid * P_PER_W

    def chunk_body(ci, carry):
        p0 = base + ci * C
        cp_pos = pltpu.async_copy(pos_hbm.at[pl.ds(p0, C)], pos_v, sem)
        cps_emb = [
            pltpu.async_copy(emb_hbm.at[b, pl.ds(p0, C)], emb_v.at[b], sem)
            for b in range(B)
        ]
        cp_pos.wait()
        for cp in cps_emb:
            cp.wait()

        def row_body(r, carry2):
            def col_body(j, carry3):
                c0 = j * L
                pv = pos_v[r, pl.ds(c0, L)]
                for b in range(B):
                    emb_v[b, r, pl.ds(c0, L)] = emb_v[b, r, pl.ds(c0, L)] + pv
                return carry3

            return lax.fori_loop(0, D_VREGS, col_body, carry2)

        lax.fori_loop(0, C, row_body, 0)

        cps_out = [
            pltpu.async_copy(emb_v.at[b], out_hbm.at[b, pl.ds(p0, C)], sem)
            for b in range(B)
        ]
        for cp in cps_out:
            cp.wait()
        return carry

    lax.fori_loop(0, N_CHUNKS, chunk_body, 0)


def kernel(embeddings, pos_table):
    return _pos_encode_sc(embeddings, pos_table)


# speedup vs baseline: 1.2943x; 1.2943x over previous
import functools

import jax
import jax.numpy as jnp
from jax import lax
from jax.experimental import pallas as pl
from jax.experimental.pallas import tpu as pltpu
from jax.experimental.pallas import tpu_sc as plsc

B, S, D = 4, 4096, 1024
NC, NS, L = 2, 16, 16
NW = NC * NS
P_PER_W = S // NW
C = 8
N_CHUNKS = P_PER_W // C
D_VREGS = D // L


_mesh = plsc.VectorSubcoreMesh(core_axis_name="c", subcore_axis_name="s")


@functools.partial(
    pl.kernel,
    mesh=_mesh,
    out_type=jax.ShapeDtypeStruct((B, S, D), jnp.float32),
    scratch_types=[
        pltpu.VMEM((C, D), jnp.float32),
        pltpu.VMEM((B, C, D), jnp.float32),
        pltpu.SemaphoreType.DMA,
    ],
)
def _pos_encode_sc(emb_hbm, pos_hbm, out_hbm, pos_v, emb_v, sem):
    wid = lax.axis_index("s") * NC + lax.axis_index("c")
    base = wid * P_PER_W

    def chunk_body(ci, carry):
        p0 = base + ci * C
        cp_pos = pltpu.async_copy(pos_hbm.at[pl.ds(p0, C)], pos_v, sem)
        cps_emb = [
            pltpu.async_copy(emb_hbm.at[b, pl.ds(p0, C)], emb_v.at[b], sem)
            for b in range(B)
        ]
        cp_pos.wait()
        for cp in cps_emb:
            cp.wait()

        def row_body(r, carry2):
            for j in range(D_VREGS):
                c0 = j * L
                pv = pos_v[r, pl.ds(c0, L)]
                for b in range(B):
                    emb_v[b, r, pl.ds(c0, L)] = emb_v[b, r, pl.ds(c0, L)] + pv
            return carry2

        lax.fori_loop(0, C, row_body, 0)

        cps_out = [
            pltpu.async_copy(emb_v.at[b], out_hbm.at[b, pl.ds(p0, C)], sem)
            for b in range(B)
        ]
        for cp in cps_out:
            cp.wait()
        return carry

    lax.fori_loop(0, N_CHUNKS, chunk_body, 0)


def kernel(embeddings, pos_table):
    return _pos_encode_sc(embeddings, pos_table)


# speedup vs baseline: 1.8261x; 1.4109x over previous
import functools

import jax
import jax.numpy as jnp
from jax import lax
from jax.experimental import pallas as pl
from jax.experimental.pallas import tpu as pltpu
from jax.experimental.pallas import tpu_sc as plsc

B, S, D = 4, 4096, 1024
NC, NS, L = 2, 16, 16
NW = NC * NS
P_PER_W = S // NW
C = 4
N_CHUNKS = P_PER_W // C
D_VREGS = D // L


_mesh = plsc.VectorSubcoreMesh(core_axis_name="c", subcore_axis_name="s")


@functools.partial(
    pl.kernel,
    mesh=_mesh,
    out_type=jax.ShapeDtypeStruct((B, S, D), jnp.float32),
    scratch_types=[
        pltpu.VMEM((2, C, D), jnp.float32),
        pltpu.VMEM((2, B, C, D), jnp.float32),
        pltpu.VMEM((2, B, C, D), jnp.float32),
        pltpu.SemaphoreType.DMA,
        pltpu.SemaphoreType.DMA,
        pltpu.SemaphoreType.DMA,
        pltpu.SemaphoreType.DMA,
    ],
)
def _pos_encode_sc(emb_hbm, pos_hbm, out_hbm, pos_v, ein_v, eout_v,
                   sem_in0, sem_in1, sem_out0, sem_out1):
    wid = lax.axis_index("s") * NC + lax.axis_index("c")
    base = wid * P_PER_W
    sems_in = (sem_in0, sem_in1)
    sems_out = (sem_out0, sem_out1)

    def issue_in(ci, k):
        p0 = base + ci * C
        pltpu.async_copy(pos_hbm.at[pl.ds(p0, C)], pos_v.at[k], sems_in[k])
        for b in range(B):
            pltpu.async_copy(emb_hbm.at[b, pl.ds(p0, C)], ein_v.at[k, b],
                             sems_in[k])

    def wait_in(k):
        pltpu.make_async_copy(pos_hbm.at[pl.ds(0, C)], pos_v.at[k],
                              sems_in[k]).wait()
        for b in range(B):
            pltpu.make_async_copy(emb_hbm.at[b, pl.ds(0, C)], ein_v.at[k, b],
                                  sems_in[k]).wait()

    def issue_out(ci, k):
        p0 = base + ci * C
        for b in range(B):
            pltpu.async_copy(eout_v.at[k, b], out_hbm.at[b, pl.ds(p0, C)],
                             sems_out[k])

    def wait_out(k):
        for b in range(B):
            pltpu.make_async_copy(eout_v.at[k, b],
                                  out_hbm.at[b, pl.ds(0, C)],
                                  sems_out[k]).wait()

    def compute(k):
        def row_body(r, carry):
            for j in range(D_VREGS):
                c0 = j * L
                pv = pos_v[k, r, pl.ds(c0, L)]
                for b in range(B):
                    eout_v[k, b, r, pl.ds(c0, L)] = (
                        ein_v[k, b, r, pl.ds(c0, L)] + pv)
            return carry

        lax.fori_loop(0, C, row_body, 0)

    issue_in(0, 0)
    issue_in(1, 1)

    def chunk_body(ci, carry):
        k = lax.rem(ci, 2)

        def run(kk):
            wait_in(kk)
            lax.cond(ci >= 2, lambda: wait_out(kk), lambda: None)
            compute(kk)
            issue_out(ci, kk)
            lax.cond(ci < N_CHUNKS - 2, lambda: issue_in(ci + 2, kk),
                     lambda: None)

        lax.cond(k == 0, lambda: run(0), lambda: run(1))
        return carry

    lax.fori_loop(0, N_CHUNKS, chunk_body, 0)

    wait_out(0)
    wait_out(1)


def kernel(embeddings, pos_table):
    return _pos_encode_sc(embeddings, pos_table)


# speedup vs baseline: 1.8561x; 1.0164x over previous
import functools

import jax
import jax.numpy as jnp
from jax import lax
from jax.experimental import pallas as pl
from jax.experimental.pallas import tpu as pltpu
from jax.experimental.pallas import tpu_sc as plsc

B, S, D = 4, 4096, 1024
NC, NS, L = 2, 16, 16
NW = NC * NS
P_PER_W = S // NW
C = 4
N_CHUNKS = P_PER_W // C
D_VREGS = D // L


_mesh = plsc.VectorSubcoreMesh(core_axis_name="c", subcore_axis_name="s")


@functools.partial(
    pl.kernel,
    mesh=_mesh,
    out_type=jax.ShapeDtypeStruct((B, S, D), jnp.float32),
    scratch_types=[
        pltpu.VMEM((2, C, D), jnp.float32),
        pltpu.VMEM((2, B, C, D), jnp.float32),
        pltpu.VMEM((2, B, C, D), jnp.float32),
        pltpu.SemaphoreType.DMA,
        pltpu.SemaphoreType.DMA,
        pltpu.SemaphoreType.DMA,
        pltpu.SemaphoreType.DMA,
    ],
)
def _pos_encode_sc(emb_hbm, pos_hbm, out_hbm, pos_v, ein_v, eout_v,
                   sem_in0, sem_in1, sem_out0, sem_out1):
    wid = lax.axis_index("s") * NC + lax.axis_index("c")
    base = wid * P_PER_W
    sems_in = (sem_in0, sem_in1)
    sems_out = (sem_out0, sem_out1)

    def issue_in(ci, k):
        p0 = base + ci * C
        pltpu.async_copy(pos_hbm.at[pl.ds(p0, C)], pos_v.at[k], sems_in[k])
        pltpu.async_copy(emb_hbm.at[:, pl.ds(p0, C)], ein_v.at[k], sems_in[k])

    def wait_in(k):
        pltpu.make_async_copy(pos_hbm.at[pl.ds(0, C)], pos_v.at[k],
                              sems_in[k]).wait()
        pltpu.make_async_copy(emb_hbm.at[:, pl.ds(0, C)], ein_v.at[k],
                              sems_in[k]).wait()

    def issue_out(ci, k):
        p0 = base + ci * C
        pltpu.async_copy(eout_v.at[k], out_hbm.at[:, pl.ds(p0, C)],
                         sems_out[k])

    def wait_out(k):
        pltpu.make_async_copy(eout_v.at[k], out_hbm.at[:, pl.ds(0, C)],
                              sems_out[k]).wait()

    def compute(k):
        def row_body(r, carry):
            for j in range(D_VREGS):
                c0 = j * L
                pv = pos_v[k, r, pl.ds(c0, L)]
                for b in range(B):
                    eout_v[k, b, r, pl.ds(c0, L)] = (
                        ein_v[k, b, r, pl.ds(c0, L)] + pv)
            return carry

        lax.fori_loop(0, C, row_body, 0)

    issue_in(0, 0)
    issue_in(1, 1)

    def chunk_body(ci, carry):
        k = lax.rem(ci, 2)

        def run(kk):
            wait_in(kk)
            lax.cond(ci >= 2, lambda: wait_out(kk), lambda: None)
            compute(kk)
            issue_out(ci, kk)
            lax.cond(ci < N_CHUNKS - 2, lambda: issue_in(ci + 2, kk),
                     lambda: None)

        lax.cond(k == 0, lambda: run(0), lambda: run(1))
        return carry

    lax.fori_loop(0, N_CHUNKS, chunk_body, 0)

    wait_out(0)
    wait_out(1)


def kernel(embeddings, pos_table):
    return _pos_encode_sc(embeddings, pos_table)
